# baseline (speedup 1.0000x reference)
import functools

import jax
import jax.numpy as jnp
from jax import lax
from jax.experimental import pallas as pl
from jax.experimental.pallas import tpu as pltpu
from jax.experimental.pallas import tpu_sc as plsc

_N_ROWS = 819200
_TABLE_ROWS = 1000000
_D = 64
_NW = 32
_ROWS_PER_DMA = 512
_CPW = _N_ROWS // (_NW * _ROWS_PER_DMA)

_mesh = plsc.VectorSubcoreMesh(core_axis_name="c", subcore_axis_name="s")


@functools.partial(
    pl.kernel,
    out_type=jax.ShapeDtypeStruct((_N_ROWS, 128), jnp.float32),
    mesh=_mesh,
    scratch_types=[
        pltpu.VMEM((_CPW, _ROWS_PER_DMA), jnp.int32),
        pltpu.VMEM((_ROWS_PER_DMA, _D), jnp.float32),
        pltpu.VMEM((_ROWS_PER_DMA, _D), jnp.float32),
        pltpu.SemaphoreType.DMA,
        pltpu.SemaphoreType.DMA,
        pltpu.SemaphoreType.DMA,
        pltpu.SemaphoreType.DMA,
    ],
    compiler_params=pltpu.CompilerParams(use_tc_tiling_on_sc=False),
)
def _gather(table_hbm, idx_hbm, out_hbm, idx_v, buf0, buf1,
            gsem0, gsem1, wsem0, wsem1):
    wid = lax.axis_index("s") * 2 + lax.axis_index("c")
    base = wid * _CPW
    pltpu.sync_copy(idx_hbm.at[pl.ds(base, _CPW)], idx_v)

    def out_slice(t):
        return out_hbm.at[pl.ds((base + t) * _ROWS_PER_DMA, _ROWS_PER_DMA),
                          pl.ds(0, _D)]

    def fire_gather(t, buf, sem):
        pltpu.async_copy(table_hbm.at[idx_v.at[t]], buf, sem)

    def wait_gather(t, buf, sem):
        pltpu.make_async_copy(table_hbm.at[idx_v.at[t]], buf, sem).wait()

    def fire_wb(t, buf, sem):
        pltpu.async_copy(buf, out_slice(t), sem)

    def wait_wb(t, buf, sem):
        pltpu.make_async_copy(buf, out_slice(t), sem).wait()

    fire_gather(0, buf0, gsem0)

    def pair(p, carry):
        t0 = 2 * p
        t1 = t0 + 1
        wait_gather(t0, buf0, gsem0)

        @pl.when(p > 0)
        def _():
            wait_wb(t0 - 1, buf1, wsem1)

        fire_gather(t1, buf1, gsem1)
        fire_wb(t0, buf0, wsem0)
        wait_gather(t1, buf1, gsem1)
        wait_wb(t0, buf0, wsem0)

        @pl.when(p < _CPW // 2 - 1)
        def _():
            fire_gather(t1 + 1, buf0, gsem0)

        fire_wb(t1, buf1, wsem1)
        return carry

    lax.fori_loop(0, _CPW // 2, pair, 0)
    wait_wb(_CPW - 1, buf1, wsem1)


def kernel(x, weight):
    wt = jnp.pad(weight, ((0, 0), (0, 64))).reshape(2 * _TABLE_ROWS, _D)
    idx = jnp.reshape(x * 2, (_N_ROWS // _ROWS_PER_DMA, _ROWS_PER_DMA))
    o = _gather(wt, idx)
    return o[:, :_D]
